# baseline (speedup 1.0000x reference)
import functools

import jax
import jax.numpy as jnp
from jax import lax
from jax.experimental import pallas as pl
from jax.experimental.pallas import tpu as pltpu
from jax.experimental.pallas import tpu_sc as plsc

BATCH = 16384
DIM = 64
NUM_MEMS = 20
MARGIN = 1.0

_NC = 2
_NS = 16
_NW = _NC * _NS
_ROWS_PER_W = BATCH // _NW


def _gather_body(uid, pid, nid, uemb, iemb, ue_out, pe_out, ne_out,
                 uidx_v, pidx_v, nidx_v, ue_v, pe_v, ne_v, s0, s1, s2):
    wid = lax.axis_index("s") * _NC + lax.axis_index("c")
    base = wid * _ROWS_PER_W
    sl = pl.ds(base, _ROWS_PER_W)
    pltpu.sync_copy(uid.at[sl], uidx_v)
    pltpu.sync_copy(pid.at[sl], pidx_v)
    pltpu.sync_copy(nid.at[sl], nidx_v)
    cu = pltpu.async_copy(uemb.at[uidx_v], ue_v, s0)
    cp = pltpu.async_copy(iemb.at[pidx_v], pe_v, s1)
    cn = pltpu.async_copy(iemb.at[nidx_v], ne_v, s2)
    cu.wait()
    pltpu.sync_copy(ue_v, ue_out.at[sl])
    cp.wait()
    pltpu.sync_copy(pe_v, pe_out.at[sl])
    cn.wait()
    pltpu.sync_copy(ne_v, ne_out.at[sl])


def _sc_gather(uid, pid, nid, uemb, iemb):
    mesh = plsc.VectorSubcoreMesh(core_axis_name="c", subcore_axis_name="s")
    f = pl.kernel(
        _gather_body,
        out_type=[jax.ShapeDtypeStruct((BATCH, DIM), jnp.float32)] * 3,
        mesh=mesh,
        scratch_types=[
            pltpu.VMEM((_ROWS_PER_W,), jnp.int32),
            pltpu.VMEM((_ROWS_PER_W,), jnp.int32),
            pltpu.VMEM((_ROWS_PER_W,), jnp.int32),
            pltpu.VMEM((_ROWS_PER_W, DIM), jnp.float32),
            pltpu.VMEM((_ROWS_PER_W, DIM), jnp.float32),
            pltpu.VMEM((_ROWS_PER_W, DIM), jnp.float32),
            pltpu.SemaphoreType.DMA,
            pltpu.SemaphoreType.DMA,
            pltpu.SemaphoreType.DMA,
        ],
        compiler_params=pltpu.CompilerParams(use_tc_tiling_on_sc=False),
    )
    return f(uid, pid, nid, uemb, iemb)


_BLK = 2048


def _compute_body(key_ref, mem_ref, ue_ref, pe_ref, ne_ref, out_ref):
    ue = ue_ref[...]
    pe = pe_ref[...]
    ne = ne_ref[...]
    s = ue * pe
    logits = jnp.dot(s, key_ref[...], preferred_element_type=jnp.float32)
    m = jnp.max(logits, axis=-1, keepdims=True)
    w = jnp.exp(logits - m)
    attn = w / jnp.sum(w, axis=-1, keepdims=True)
    lat = jnp.dot(attn, mem_ref[...], preferred_element_type=jnp.float32)
    diff = ue + lat
    pos_d = jnp.sum(jnp.square(diff - pe), axis=-1)
    neg_d = jnp.sum(jnp.square(diff - ne), axis=-1)
    blk = jnp.sum(jnp.maximum(MARGIN + pos_d - neg_d, 0.0))

    @pl.when(pl.program_id(0) == 0)
    def _():
        out_ref[0, 0] = 0.0

    out_ref[0, 0] += blk


def _tc_compute(ue, pe, ne, user_item_key, memories):
    grid = BATCH // _BLK
    emb_spec = pl.BlockSpec((_BLK, DIM), lambda i: (i, 0))
    out = pl.pallas_call(
        _compute_body,
        grid=(grid,),
        in_specs=[
            pl.BlockSpec((DIM, NUM_MEMS), lambda i: (0, 0)),
            pl.BlockSpec((NUM_MEMS, DIM), lambda i: (0, 0)),
            emb_spec, emb_spec, emb_spec,
        ],
        out_specs=pl.BlockSpec(memory_space=pltpu.SMEM),
        out_shape=jax.ShapeDtypeStruct((1, 1), jnp.float32),
    )(user_item_key, memories, ue, pe, ne)
    return out[0, 0]


def kernel(user_ids, pos_ids, neg_ids, user_emb, item_emb, user_item_key, memories):
    uid = user_ids.astype(jnp.int32)
    pid = pos_ids.astype(jnp.int32)
    nid = neg_ids.astype(jnp.int32)
    ue, pe, ne = _sc_gather(uid, pid, nid, user_emb, item_emb)
    return _tc_compute(ue, pe, ne, user_item_key, memories)
